# Initial kernel scaffold; baseline (speedup 1.0000x reference)
import jax
import jax.numpy as jnp
from jax.experimental import pallas as pl


def kernel(y, idx_map):
    raise NotImplementedError("write your pallas kernel here")


# speedup vs baseline: 1.1031x; 1.1031x over previous
import functools

import jax
import jax.numpy as jnp
from jax import lax
from jax.experimental import pallas as pl
from jax.experimental.pallas import tpu as pltpu
from jax.experimental.pallas import tpu_sc as plsc

_NC = 2
_NS = 16
_NW = _NC * _NS


def _gather_sc(y, idx_map):
    batch = y.shape[0]
    bpw = batch // _NW

    mesh = plsc.VectorSubcoreMesh(core_axis_name="c", subcore_axis_name="s")

    @functools.partial(
        pl.kernel,
        out_type=jax.ShapeDtypeStruct((batch,), jnp.int32),
        mesh=mesh,
        scratch_types=[
            pltpu.VMEM((bpw,), jnp.int32),
            pltpu.VMEM((bpw,), jnp.int32),
            pltpu.SemaphoreType.DMA,
        ],
    )
    def k(y_hbm, table_hbm, out_hbm, idx_v, vals_v, sem):
        wid = lax.axis_index("s") * _NC + lax.axis_index("c")
        base = wid * bpw
        pltpu.sync_copy(y_hbm.at[pl.ds(base, bpw)], idx_v)
        pltpu.async_copy(table_hbm.at[idx_v], vals_v, sem).wait()
        pltpu.sync_copy(vals_v, out_hbm.at[pl.ds(base, bpw)])

    return k(y, idx_map)


def kernel(y, idx_map):
    return _gather_sc(y, idx_map)


# speedup vs baseline: 1.1112x; 1.0074x over previous
import functools

import jax
import jax.numpy as jnp
from jax import lax
from jax.experimental import pallas as pl
from jax.experimental.pallas import tpu as pltpu
from jax.experimental.pallas import tpu_sc as plsc

_NC = 2
_NS = 16
_NW = _NC * _NS


_NCHUNK = 4


def _gather_sc(y, idx_map):
    batch = y.shape[0]
    bpw = batch // _NW
    ch = bpw // _NCHUNK

    mesh = plsc.VectorSubcoreMesh(core_axis_name="c", subcore_axis_name="s")

    @functools.partial(
        pl.kernel,
        out_type=jax.ShapeDtypeStruct((batch,), jnp.int32),
        mesh=mesh,
        scratch_types=[
            pltpu.VMEM((bpw,), jnp.int32),
            pltpu.VMEM((bpw,), jnp.int32),
        ]
        + [pltpu.SemaphoreType.DMA] * (2 * _NCHUNK + 1),
    )
    def k(y_hbm, table_hbm, out_hbm, idx_v, vals_v, *sems):
        sem_i = sems[:_NCHUNK]
        sem_g = sems[_NCHUNK:2 * _NCHUNK]
        sem_o = sems[2 * _NCHUNK]
        wid = lax.axis_index("s") * _NC + lax.axis_index("c")
        base = wid * bpw
        for c in range(_NCHUNK):
            pltpu.async_copy(
                y_hbm.at[pl.ds(base + c * ch, ch)],
                idx_v.at[pl.ds(c * ch, ch)],
                sem_i[c],
            )
        for c in range(_NCHUNK):
            pltpu.make_async_copy(
                y_hbm.at[pl.ds(base + c * ch, ch)],
                idx_v.at[pl.ds(c * ch, ch)],
                sem_i[c],
            ).wait()
            pltpu.async_copy(
                table_hbm.at[idx_v.at[pl.ds(c * ch, ch)]],
                vals_v.at[pl.ds(c * ch, ch)],
                sem_g[c],
            )
        for c in range(_NCHUNK):
            pltpu.make_async_copy(
                table_hbm.at[idx_v.at[pl.ds(c * ch, ch)]],
                vals_v.at[pl.ds(c * ch, ch)],
                sem_g[c],
            ).wait()
            pltpu.async_copy(
                vals_v.at[pl.ds(c * ch, ch)],
                out_hbm.at[pl.ds(base + c * ch, ch)],
                sem_o,
            )
        for c in range(_NCHUNK):
            pltpu.make_async_copy(
                vals_v.at[pl.ds(c * ch, ch)],
                out_hbm.at[pl.ds(base + c * ch, ch)],
                sem_o,
            ).wait()

    return k(y, idx_map)


def kernel(y, idx_map):
    return _gather_sc(y, idx_map)


# speedup vs baseline: 1.1266x; 1.0138x over previous
import functools

import jax
import jax.numpy as jnp
from jax import lax
from jax.experimental import pallas as pl
from jax.experimental.pallas import tpu as pltpu
from jax.experimental.pallas import tpu_sc as plsc

_NC = 2
_NS = 16
_NW = _NC * _NS


def _gather_sc(y, idx_map):
    batch = y.shape[0]
    vocab = idx_map.shape[0]
    bpw = batch // _NW

    mesh = plsc.VectorSubcoreMesh(core_axis_name="c", subcore_axis_name="s")

    @functools.partial(
        pl.kernel,
        out_type=jax.ShapeDtypeStruct((batch,), jnp.int32),
        mesh=mesh,
        scratch_types=[
            pltpu.VMEM((bpw,), jnp.int32),
            pltpu.VMEM((bpw,), jnp.int32),
            pltpu.VMEM_SHARED((vocab,), jnp.int32),
            pltpu.SemaphoreType.DMA,
            pltpu.SemaphoreType.DMA,
            pltpu.SemaphoreType.DMA,
        ],
    )
    def k(y_hbm, table_hbm, out_hbm, idx_v, vals_v, tab_s, sem_i, sem_t, sem_g):
        sid = lax.axis_index("s")
        wid = sid * _NC + lax.axis_index("c")
        base = wid * bpw
        pltpu.async_copy(y_hbm.at[pl.ds(base, bpw)], idx_v, sem_i)

        @pl.when(sid == 0)
        def _():
            pltpu.async_copy(table_hbm, tab_s, sem_t).wait()

        plsc.subcore_barrier()
        pltpu.make_async_copy(
            y_hbm.at[pl.ds(base, bpw)], idx_v, sem_i
        ).wait()
        pltpu.async_copy(tab_s.at[idx_v], vals_v, sem_g).wait()
        pltpu.sync_copy(vals_v, out_hbm.at[pl.ds(base, bpw)])

    return k(y, idx_map)


def kernel(y, idx_map):
    return _gather_sc(y, idx_map)


# speedup vs baseline: 1.1300x; 1.0030x over previous
import functools

import jax
import jax.numpy as jnp
from jax import lax
from jax.experimental import pallas as pl
from jax.experimental.pallas import tpu as pltpu
from jax.experimental.pallas import tpu_sc as plsc

_NC = 2
_NS = 16
_NW = _NC * _NS


def _gather_sc(y, idx_map):
    batch = y.shape[0]
    vocab = idx_map.shape[0]
    bpw = batch // _NW

    mesh = plsc.VectorSubcoreMesh(core_axis_name="c", subcore_axis_name="s")

    nhbm = 128
    nch = 2
    ch = (bpw - nhbm) // nch

    @functools.partial(
        pl.kernel,
        out_type=jax.ShapeDtypeStruct((batch,), jnp.int32),
        mesh=mesh,
        scratch_types=[
            pltpu.VMEM((bpw,), jnp.int32),
            pltpu.VMEM((bpw,), jnp.int32),
            pltpu.VMEM_SHARED((vocab,), jnp.int32),
            pltpu.SemaphoreType.DMA,
            pltpu.SemaphoreType.DMA,
            pltpu.SemaphoreType.DMA,
            pltpu.SemaphoreType.DMA,
        ]
        + [pltpu.SemaphoreType.DMA] * nch,
    )
    def k(y_hbm, table_hbm, out_hbm, idx_v, vals_v, tab_s,
          sem_i, sem_t, sem_h, sem_o, *sem_g):
        sid = lax.axis_index("s")
        wid = sid * _NC + lax.axis_index("c")
        base = wid * bpw
        pltpu.async_copy(y_hbm.at[pl.ds(base, bpw)], idx_v, sem_i)

        @pl.when(sid == 0)
        def _():
            pltpu.async_copy(table_hbm, tab_s, sem_t)

        pltpu.make_async_copy(y_hbm.at[pl.ds(base, bpw)], idx_v, sem_i).wait()
        pltpu.async_copy(
            table_hbm.at[idx_v.at[pl.ds(0, nhbm)]],
            vals_v.at[pl.ds(0, nhbm)],
            sem_h,
        )

        @pl.when(sid == 0)
        def _():
            pltpu.make_async_copy(table_hbm, tab_s, sem_t).wait()

        plsc.subcore_barrier()
        for c in range(nch):
            pltpu.async_copy(
                tab_s.at[idx_v.at[pl.ds(nhbm + c * ch, ch)]],
                vals_v.at[pl.ds(nhbm + c * ch, ch)],
                sem_g[c],
            )
        pltpu.make_async_copy(
            table_hbm.at[idx_v.at[pl.ds(0, nhbm)]],
            vals_v.at[pl.ds(0, nhbm)],
            sem_h,
        ).wait()
        pltpu.async_copy(
            vals_v.at[pl.ds(0, nhbm)], out_hbm.at[pl.ds(base, nhbm)], sem_o
        )
        for c in range(nch):
            pltpu.make_async_copy(
                tab_s.at[idx_v.at[pl.ds(nhbm + c * ch, ch)]],
                vals_v.at[pl.ds(nhbm + c * ch, ch)],
                sem_g[c],
            ).wait()
            pltpu.async_copy(
                vals_v.at[pl.ds(nhbm + c * ch, ch)],
                out_hbm.at[pl.ds(base + nhbm + c * ch, ch)],
                sem_o,
            )
        pltpu.make_async_copy(
            vals_v.at[pl.ds(0, nhbm)], out_hbm.at[pl.ds(base, nhbm)], sem_o
        ).wait()
        for c in range(nch):
            pltpu.make_async_copy(
                vals_v.at[pl.ds(nhbm + c * ch, ch)],
                out_hbm.at[pl.ds(base + nhbm + c * ch, ch)],
                sem_o,
            ).wait()

    return k(y, idx_map)


def kernel(y, idx_map):
    return _gather_sc(y, idx_map)
